# baseline (speedup 1.0000x reference)
import jax
import jax.numpy as jnp
from jax import lax
from jax.experimental import pallas as pl
from jax.experimental.pallas import tpu as pltpu
from jax.experimental.pallas import tpu_sc as plsc

_B, _J, _C = 4096, 25, 256
_REG = ((0, 0), (1, 20), (3, 3), (2, 20), (21, 21), (22, 7), (6, 5),
        (4, 20), (23, 23), (24, 11), (10, 9), (8, 20), (0, 0), (12, 13),
        (14, 15), (0, 0), (16, 17), (18, 19))
_R = len(_REG)

_SING = tuple((r, a) for r, (a, b) in enumerate(_REG) if a == b)
_PAIRS = tuple((r, a, b) for r, (a, b) in enumerate(_REG) if a != b)
_NP = len(_PAIRS)

_CJ = tuple(sorted({j for _, a, b in _PAIRS for j in (a, b)}))
_NJ = len(_CJ)
_JMAP = {j: i for i, j in enumerate(_CJ)}


def _runs(seq):
    out, s, p = [], seq[0], seq[0]
    for v in seq[1:]:
        if v != p + 1:
            out.append((s, p - s + 1))
            s = v
        p = v
    out.append((s, p - s + 1))
    return tuple(out)


_IN_RUNS = _runs(_CJ)
_OUT_RUNS = _runs([r for r, _, _ in _PAIRS])

_NC, _NS = 2, 16
_NW = _NC * _NS
_BW = _B // _NW
_CB = 8
_NCHUNK = _BW // _CB
_LANES = 16
_HC = _C // 2
_NLG = _HC // _LANES


def _body(x_hbm, o_hbm, in0, in1, out0, out1,
          isem0, isem1, osem0, osem1, ssem):
    wid = lax.axis_index("s") * _NC + lax.axis_index("c")
    start = wid * _BW
    ins, outs, isems, osems = (in0, in1), (out0, out1), (isem0, isem1), (osem0, osem1)

    def sing_copies():
        cps = []
        for r, j in _SING:
            cps.append(pltpu.make_async_copy(
                x_hbm.at[j, pl.ds(start, _BW), :],
                o_hbm.at[r, pl.ds(start, _BW), :], ssem))
        return cps

    def in_copies(c, half, slot):
        cps, row = [], 0
        for js, jl in _IN_RUNS:
            cps.append(pltpu.make_async_copy(
                x_hbm.at[pl.ds(js, jl), pl.ds(start + c * _CB, _CB),
                         pl.ds(half * _HC, _HC)],
                ins[slot].at[pl.ds(row, jl)], isems[slot]))
            row += jl
        return cps

    def out_copies(c, half, slot):
        cps, row = [], 0
        for rs, rl in _OUT_RUNS:
            cps.append(pltpu.make_async_copy(
                outs[slot].at[pl.ds(row, rl)],
                o_hbm.at[pl.ds(rs, rl), pl.ds(start + c * _CB, _CB),
                         pl.ds(half * _HC, _HC)], osems[slot]))
            row += rl
        return cps

    def start_all(cps):
        for cp in cps:
            cp.start()

    def wait_all(cps):
        for cp in cps:
            cp.wait()

    def compute(slot):
        in_v, out_v = ins[slot], outs[slot]

        def batch(b, carry):
            for lg in range(_NLG):
                s = lg * _LANES
                rows = [in_v[i, b, pl.ds(s, _LANES)] for i in range(_NJ)]
                for p, (r, a, bj) in enumerate(_PAIRS):
                    out_v[p, b, pl.ds(s, _LANES)] = (
                        rows[_JMAP[a]] + rows[_JMAP[bj]]) * 0.5
            return carry

        lax.fori_loop(0, _CB, batch, 0)

    scps = sing_copies()
    start_all(scps)
    start_all(in_copies(0, 0, 0))

    def chunk(c, carry):
        for half in range(2):
            slot = half
            nxt = 1 - half
            if half == 0:
                start_all(in_copies(c, 1, nxt))
            else:
                @pl.when(c + 1 < _NCHUNK)
                def _():
                    start_all(in_copies(c + 1, 0, nxt))

            wait_all(in_copies(c, half, slot))

            @pl.when(2 * c + half >= 2)
            def _():
                pc = c - 1 + half
                wait_all(out_copies(pc, half, slot))

            compute(slot)
            start_all(out_copies(c, half, slot))
        return carry

    lax.fori_loop(0, _NCHUNK, chunk, 0)
    wait_all(out_copies(_NCHUNK - 1, 0, 0))
    wait_all(out_copies(_NCHUNK - 1, 1, 1))
    wait_all(scps)


@jax.jit
def kernel(x):
    xt = jnp.transpose(x, (1, 0, 2))
    mesh = plsc.VectorSubcoreMesh(core_axis_name="c", subcore_axis_name="s")
    f = pl.kernel(
        _body,
        out_type=jax.ShapeDtypeStruct((_R, _B, _C), jnp.float32),
        mesh=mesh,
        scratch_types=[
            pltpu.VMEM((_NJ, _CB, _HC), jnp.float32),
            pltpu.VMEM((_NJ, _CB, _HC), jnp.float32),
            pltpu.VMEM((_NP, _CB, _HC), jnp.float32),
            pltpu.VMEM((_NP, _CB, _HC), jnp.float32),
            pltpu.SemaphoreType.DMA,
            pltpu.SemaphoreType.DMA,
            pltpu.SemaphoreType.DMA,
            pltpu.SemaphoreType.DMA,
            pltpu.SemaphoreType.DMA,
        ],
    )
    ot = f(xt)
    return jnp.transpose(ot, (1, 0, 2))


# speedup vs baseline: 8.9668x; 8.9668x over previous
import jax
import jax.numpy as jnp
from jax import lax
from jax.experimental import pallas as pl
from jax.experimental.pallas import tpu as pltpu
from jax.experimental.pallas import tpu_sc as plsc

_B, _J, _C = 4096, 25, 256
_REG = ((0, 0), (1, 20), (3, 3), (2, 20), (21, 21), (22, 7), (6, 5),
        (4, 20), (23, 23), (24, 11), (10, 9), (8, 20), (0, 0), (12, 13),
        (14, 15), (0, 0), (16, 17), (18, 19))
_R = len(_REG)
_NC, _NS = 2, 16
_NW = _NC * _NS
_BW = _B // _NW
_CB = 8
_NCHUNK = _BW // _CB
_LANES = 16
_HC = _C // 2
_NLG = _HC // _LANES


def _body(x_hbm, o_hbm, in0, in1, out0, out1, isem0, isem1, osem0, osem1):
    wid = lax.axis_index("s") * _NC + lax.axis_index("c")
    start = wid * _BW
    ins, outs, isems, osems = (in0, in1), (out0, out1), (isem0, isem1), (osem0, osem1)

    def in_copy(c, half, slot):
        return pltpu.make_async_copy(
            x_hbm.at[:, pl.ds(start + c * _CB, _CB), pl.ds(half * _HC, _HC)],
            ins[slot], isems[slot])

    def out_copy(c, half, slot):
        return pltpu.make_async_copy(
            outs[slot],
            o_hbm.at[:, pl.ds(start + c * _CB, _CB), pl.ds(half * _HC, _HC)],
            osems[slot])

    def compute(slot):
        in_v, out_v = ins[slot], outs[slot]

        def batch(b, carry):
            for lg in range(_NLG):
                s = lg * _LANES
                rows = [in_v[j, b, pl.ds(s, _LANES)] for j in range(_J)]
                for r in range(_R):
                    j0, j1 = _REG[r]
                    if j0 == j1:
                        out_v[r, b, pl.ds(s, _LANES)] = rows[j0]
                    else:
                        out_v[r, b, pl.ds(s, _LANES)] = (rows[j0] + rows[j1]) * 0.5
            return carry

        lax.fori_loop(0, _CB, batch, 0)

    in_copy(0, 0, 0).start()

    def chunk(c, carry):
        for half in range(2):
            slot = half
            nxt = 1 - half
            if half == 0:
                in_copy(c, 1, nxt).start()
            else:
                @pl.when(c + 1 < _NCHUNK)
                def _():
                    in_copy(c + 1, 0, nxt).start()

            in_copy(c, half, slot).wait()

            @pl.when(2 * c + half >= 2)
            def _():
                pc = c - 1 + half
                out_copy(pc, half, slot).wait()

            compute(slot)
            out_copy(c, half, slot).start()
        return carry

    lax.fori_loop(0, _NCHUNK, chunk, 0)
    out_copy(_NCHUNK - 1, 0, 0).wait()
    out_copy(_NCHUNK - 1, 1, 1).wait()


@jax.jit
def kernel(x):
    xt = jnp.transpose(x, (1, 0, 2))
    mesh = plsc.VectorSubcoreMesh(core_axis_name="c", subcore_axis_name="s")
    f = pl.kernel(
        _body,
        out_type=jax.ShapeDtypeStruct((_R, _B, _C), jnp.float32),
        mesh=mesh,
        scratch_types=[
            pltpu.VMEM((_J, _CB, _HC), jnp.float32),
            pltpu.VMEM((_J, _CB, _HC), jnp.float32),
            pltpu.VMEM((_R, _CB, _HC), jnp.float32),
            pltpu.VMEM((_R, _CB, _HC), jnp.float32),
            pltpu.SemaphoreType.DMA,
            pltpu.SemaphoreType.DMA,
            pltpu.SemaphoreType.DMA,
            pltpu.SemaphoreType.DMA,
        ],
    )
    ot = f(xt)
    return jnp.transpose(ot, (1, 0, 2))
